# Initial kernel scaffold; baseline (speedup 1.0000x reference)
import jax
import jax.numpy as jnp
from jax.experimental import pallas as pl


def kernel(source_sentences, positions, emb_table, pos_table, W, b):
    raise NotImplementedError("write your pallas kernel here")


# speedup vs baseline: 1.4738x; 1.4738x over previous
import functools

import jax
import jax.numpy as jnp
from jax import lax
from jax.experimental import pallas as pl
from jax.experimental.pallas import tpu as pltpu
from jax.experimental.pallas import tpu_sc as plsc

EMB = 128
B, L = 1024, 50
BL = B * L
CAT = 2 * EMB
HID = 256
NW = 32
PER_W = BL // NW
CHUNK = 400
NCHUNK = PER_W // CHUNK

@functools.lru_cache(maxsize=None)
def _make_sc_gather():
    mesh = plsc.VectorSubcoreMesh(core_axis_name="c", subcore_axis_name="s")

    @functools.partial(
        pl.kernel,
        out_type=jax.ShapeDtypeStruct((BL, CAT), jnp.float32),
        mesh=mesh,
        scratch_types=[
            pltpu.VMEM((CHUNK,), jnp.int32),
            pltpu.VMEM((CHUNK,), jnp.int32),
            pltpu.VMEM((CHUNK, EMB), jnp.float32),
            pltpu.VMEM((CHUNK, EMB), jnp.float32),
            pltpu.SemaphoreType.DMA,
            pltpu.SemaphoreType.DMA,
        ],
    )
    def _sc_gather(src_hbm, pos_hbm, emb_hbm, ptab_hbm, cat_hbm,
                   sidx_v, pidx_v, wrows_v, prows_v, sem_w, sem_p):
        wid = lax.axis_index("s") * 2 + lax.axis_index("c")
        base0 = wid * PER_W
        for i in range(NCHUNK):
            base = base0 + i * CHUNK
            pltpu.sync_copy(src_hbm.at[pl.ds(base, CHUNK)], sidx_v)
            pltpu.sync_copy(pos_hbm.at[pl.ds(base, CHUNK)], pidx_v)
            w_cp = pltpu.async_copy(emb_hbm.at[sidx_v], wrows_v, sem_w)
            p_cp = pltpu.async_copy(ptab_hbm.at[pidx_v], prows_v, sem_p)
            w_cp.wait()
            pltpu.sync_copy(wrows_v, cat_hbm.at[pl.ds(base, CHUNK), pl.ds(0, EMB)])
            p_cp.wait()
            pltpu.sync_copy(prows_v, cat_hbm.at[pl.ds(base, CHUNK), pl.ds(EMB, EMB)])

    return _sc_gather


def _tc_body(cat_ref, w_ref, b_ref, out_ref):
    x = cat_ref[...]
    s = jnp.sum(x, axis=1) * (1.0 / L)
    h = lax.dot_general(s, w_ref[...], (((1,), (1,)), ((), ())),
                        preferred_element_type=jnp.float32)
    out_ref[...] = h + b_ref[...]


_BB = 128


def _tc_linear(cat3, W, b2):
    return pl.pallas_call(
        _tc_body,
        grid=(B // _BB,),
        in_specs=[
            pl.BlockSpec((_BB, L, CAT), lambda i: (i, 0, 0)),
            pl.BlockSpec((HID, CAT), lambda i: (0, 0)),
            pl.BlockSpec((1, HID), lambda i: (0, 0)),
        ],
        out_specs=pl.BlockSpec((_BB, HID), lambda i: (i, 0)),
        out_shape=jax.ShapeDtypeStruct((B, HID), jnp.float32),
    )(cat3, W, b2)


def kernel(source_sentences, positions, emb_table, pos_table, W, b):
    src = source_sentences.reshape(BL)
    posf = positions.reshape(BL)
    cat2d = _make_sc_gather()(src, posf, emb_table, pos_table)
    cat = cat2d.reshape(B, L, CAT)
    hidden = _tc_linear(cat, W, b.reshape(1, HID))
    h0 = hidden[None]
    return (cat, h0, h0)
